# baseline (speedup 1.0000x reference)
import jax
import jax.numpy as jnp
from jax import lax
from jax.experimental import pallas as pl
from jax.experimental.pallas import tpu as pltpu
from jax.experimental.pallas import tpu_sc as plsc

_N = 10000
_E = 20000
_NNZ = 320000
_H = 4
_C = 32
_HC = _H * _C
_HALF = _HC // 2
_NCORES = 2
_NSUB = 16
_CHUNK = 128
_NCH = 160
_NB = 4
_NSTAGE = 4
_CPS = _NCH // _NSTAGE
_GPS = _CPS // _NB
_PPS = _NCH * _CHUNK
_PADNNZ = _NSUB * _PPS
_W80 = _HALF + 16
_f32 = jnp.float32

_HIGH = lax.Precision.HIGHEST


def _mm_body(x_ref, w_ref, o_ref):
    o_ref[...] = lax.dot_general(
        x_ref[...], w_ref[...], (((1,), (1,)), ((), ())),
        preferred_element_type=_f32, precision=_HIGH)


def _phase_a(X, W):
    bn = 2000
    nb = _N // bn
    return pl.pallas_call(
        _mm_body,
        grid=(_NCORES, nb),
        in_specs=[
            pl.BlockSpec((bn, _HC), lambda c, i: (i, 0)),
            pl.BlockSpec((_HALF, _HC), lambda c, i: (c, 0)),
        ],
        out_specs=pl.BlockSpec((bn, _HALF), lambda c, i, _nb=nb: (c * _nb + i, 0)),
        out_shape=jax.ShapeDtypeStruct((_NCORES * _N, _HALF), _f32),
    )(X, W)


def _sliced_copy(s, total, src, dst, src_off=0, dst_off=0):
    base = (total // _NSUB) // 8 * 8
    tail = total - _NSUB * base

    def cp(r0, nrows):
        so = pl.multiple_of(src_off + r0, 8)
        do = pl.multiple_of(dst_off + r0, 8)
        pltpu.sync_copy(src.at[pl.ds(so, nrows)], dst.at[pl.ds(do, nrows)])

    cp(s * base, base)
    if tail:
        @pl.when(s == _NSUB - 1)
        def _():
            cp(_NSUB * base, tail)


def _phase_b2_body(eidx, z16, ones16, cnt_out, eall, obuf, csem, cnt_sh):
    c = lax.axis_index("c")
    s = lax.axis_index("s")
    w = c * _NSUB + s
    _sliced_copy(s, _E, z16, cnt_sh)
    pltpu.sync_copy(ones16, obuf)
    plsc.subcore_barrier()

    def wait_cnt(j):
        pltpu.make_async_copy(obuf, cnt_sh.at[eall.at[j]], csem).wait()

    for st in range(2):
        pltpu.sync_copy(eidx.at[w].at[pl.ds(st * _CPS, _CPS)], eall)

        def chunk(j, carry):
            pltpu.async_copy(obuf, cnt_sh.at[eall.at[j]], csem, add=True)

            @pl.when(j >= _NB)
            def _():
                wait_cnt(j - _NB)
            return carry

        lax.fori_loop(0, _CPS, chunk, 0)
        for j in range(_CPS - _NB, _CPS):
            wait_cnt(j)

    plsc.subcore_barrier()
    _sliced_copy(s, _E, cnt_sh, cnt_out, dst_off=c * _E)


def _phase_b2(eidx2, z16, ones16):
    mesh = plsc.VectorSubcoreMesh(core_axis_name="c", subcore_axis_name="s")
    f = pl.kernel(
        _phase_b2_body,
        out_type=jax.ShapeDtypeStruct((_NCORES * _E, 16), _f32),
        mesh=mesh,
        scratch_types=[
            pltpu.VMEM((_CPS, _CHUNK), jnp.int32),
            pltpu.VMEM((_CHUNK, 16), _f32),
            pltpu.SemaphoreType.DMA,
            pltpu.VMEM_SHARED((_E + 8, 16), _f32),
        ],
        compiler_params=pltpu.CompilerParams(use_tc_tiling_on_sc=False),
    )
    return f(eidx2, z16, ones16)


def _phase_b_body(x0f, vidx, eidx, z64,
                  xe_out,
                  vall, eall, rb0, rb1, rb2, rb3,
                  g0, g1, g2, g3, s0, s1, s2, s3,
                  xe_sh):
    c = lax.axis_index("c")
    s = lax.axis_index("s")
    w = c * _NSUB + s
    rbufs = [rb0, rb1, rb2, rb3]
    gsems = [g0, g1, g2, g3]
    ssems = [s0, s1, s2, s3]

    _sliced_copy(s, _E, z64, xe_sh)
    plsc.subcore_barrier()

    def fire_gather(j, b):
        pltpu.async_copy(x0f.at[vall.at[j]], rbufs[b], gsems[b])

    def wait_gather(j, b):
        pltpu.make_async_copy(x0f.at[vall.at[j]], rbufs[b], gsems[b]).wait()

    def step(i, b, last):
        j = i * _NB + b
        wait_gather(j, b)
        sc = pltpu.async_copy(rbufs[b], xe_sh.at[eall.at[j]], ssems[b],
                              add=True)
        sc.wait()
        if not last:
            fire_gather(j + _NB, b)

    for st in range(_NSTAGE):
        pltpu.sync_copy(vidx.at[w].at[pl.ds(st * _CPS, _CPS)], vall)
        pltpu.sync_copy(eidx.at[s].at[pl.ds(st * _CPS, _CPS)], eall)
        for b in range(_NB):
            fire_gather(b, b)

        def group(i, carry):
            for b in range(_NB):
                step(i, b, False)
            return carry

        lax.fori_loop(0, _GPS - 1, group, 0)
        for b in range(_NB):
            step(_GPS - 1, b, True)

    plsc.subcore_barrier()
    _sliced_copy(s, _E, xe_sh, xe_out, dst_off=c * _E)


def _phase_b(x0f, vidx, eidx, z64):
    mesh = plsc.VectorSubcoreMesh(core_axis_name="c", subcore_axis_name="s")
    f = pl.kernel(
        _phase_b_body,
        out_type=jax.ShapeDtypeStruct((_NCORES * _E, _HALF), _f32),
        mesh=mesh,
        scratch_types=(
            [pltpu.VMEM((_CPS, _CHUNK), jnp.int32)] * 2
            + [pltpu.VMEM((_CHUNK, _HALF), _f32)] * _NB
            + [pltpu.SemaphoreType.DMA] * (2 * _NB)
            + [pltpu.VMEM_SHARED((_E + 8, _HALF), _f32)]
        ),
        compiler_params=pltpu.CompilerParams(use_tc_tiling_on_sc=False),
    )
    return f(x0f, vidx, eidx, z64)


def _phase_c_body(xe_ref, cnt0_ref, cnt1_ref, a_ref, m2_ref, ge_ref):
    xs = xe_ref[...]
    cnt = cnt0_ref[:, 0:1] + cnt1_ref[:, 0:1]
    xe = xs / jnp.maximum(cnt, 1.0)
    al = lax.dot_general(xe, a_ref[0], (((1,), (0,)), ((), ())),
                         preferred_element_type=_f32, precision=_HIGH)
    lr = jnp.where(al >= 0.0, al, al * 0.01)
    g = jnp.exp(lr)
    gb = lax.dot_general(g, m2_ref[...], (((1,), (0,)), ((), ())),
                         preferred_element_type=_f32, precision=_HIGH)
    ge_ref[...] = jnp.concatenate([gb * xe, g[:, :16]], axis=1)


def _phase_c(xe_sum, cnt, A, M2):
    be = 2000
    nb = _E // be
    return pl.pallas_call(
        _phase_c_body,
        grid=(_NCORES, nb),
        in_specs=[
            pl.BlockSpec((be, _HALF), lambda c, i, _nb=nb: (c * _nb + i, 0)),
            pl.BlockSpec((be, 16), lambda c, i: (i, 0)),
            pl.BlockSpec((be, 16), lambda c, i, _nb=nb: (_nb + i, 0)),
            pl.BlockSpec((1, _HALF, _HALF), lambda c, i: (c, 0, 0)),
            pl.BlockSpec((_HALF, _HALF), lambda c, i: (0, 0)),
        ],
        out_specs=pl.BlockSpec((be, _W80), lambda c, i, _nb=nb: (c * _nb + i, 0)),
        out_shape=jax.ShapeDtypeStruct((_NCORES * _E, _W80), _f32),
    )(xe_sum, cnt, cnt, A, M2)


def _phase_d_body(gef, vidx, eidx, z80,
                  xn_out,
                  vall, eall, rb0, rb1, rb2, rb3,
                  g0, g1, g2, g3, s0, s1, s2, s3,
                  xn_sh):
    c = lax.axis_index("c")
    s = lax.axis_index("s")
    w = c * _NSUB + s
    rbufs = [rb0, rb1, rb2, rb3]
    gsems = [g0, g1, g2, g3]
    ssems = [s0, s1, s2, s3]

    _sliced_copy(s, _N, z80, xn_sh)
    plsc.subcore_barrier()

    def fire_gather(j, b):
        pltpu.async_copy(gef.at[eall.at[j]], rbufs[b], gsems[b])

    def wait_gather(j, b):
        pltpu.make_async_copy(gef.at[eall.at[j]], rbufs[b], gsems[b]).wait()

    def step(i, b, last):
        j = i * _NB + b
        wait_gather(j, b)
        sc = pltpu.async_copy(rbufs[b], xn_sh.at[vall.at[j]], ssems[b],
                              add=True)
        sc.wait()
        if not last:
            fire_gather(j + _NB, b)

    for st in range(_NSTAGE):
        pltpu.sync_copy(eidx.at[w].at[pl.ds(st * _CPS, _CPS)], eall)
        pltpu.sync_copy(vidx.at[s].at[pl.ds(st * _CPS, _CPS)], vall)
        for b in range(_NB):
            fire_gather(b, b)

        def group(i, carry):
            for b in range(_NB):
                step(i, b, False)
            return carry

        lax.fori_loop(0, _GPS - 1, group, 0)
        for b in range(_NB):
            step(_GPS - 1, b, True)

    plsc.subcore_barrier()
    _sliced_copy(s, _N, xn_sh, xn_out, dst_off=c * _N)


def _phase_d(gef, vidx, eidx, z80):
    mesh = plsc.VectorSubcoreMesh(core_axis_name="c", subcore_axis_name="s")
    f = pl.kernel(
        _phase_d_body,
        out_type=jax.ShapeDtypeStruct((_NCORES * _N, _W80), _f32),
        mesh=mesh,
        scratch_types=(
            [pltpu.VMEM((_CPS, _CHUNK), jnp.int32)] * 2
            + [pltpu.VMEM((_CHUNK, _W80), _f32)] * _NB
            + [pltpu.SemaphoreType.DMA] * (2 * _NB)
            + [pltpu.VMEM_SHARED((_N + 8, _W80), _f32)]
        ),
        compiler_params=pltpu.CompilerParams(use_tc_tiling_on_sc=False),
    )
    return f(gef, vidx, eidx, z80)


def _phase_e_body(xn0, xn1, x00, x01, msa, msb, o_ref):
    denb = (lax.dot_general(xn0[:, _HALF:], msa[...], (((1,), (0,)), ((), ())),
                            preferred_element_type=_f32, precision=_HIGH)
            + lax.dot_general(xn1[:, _HALF:], msb[...], (((1,), (0,)), ((), ())),
                              preferred_element_type=_f32, precision=_HIGH))
    num = jnp.concatenate([xn0[:, :_HALF], xn1[:, :_HALF]], axis=1)
    x0 = jnp.concatenate([x00[...], x01[...]], axis=1)
    o_ref[...] = num / (denb + 1e-16) + x0


def _phase_e(xn, x0f, MselA, MselB):
    bn = 2000
    nb = _N // bn

    def lo(i):
        return (i, 0)

    def hi(i, _nb=nb):
        return (_nb + i, 0)

    return pl.pallas_call(
        _phase_e_body,
        grid=(nb,),
        in_specs=[
            pl.BlockSpec((bn, _W80), lo),
            pl.BlockSpec((bn, _W80), hi),
            pl.BlockSpec((bn, _HALF), lo),
            pl.BlockSpec((bn, _HALF), hi),
            pl.BlockSpec((16, _HC), lambda i: (0, 0)),
            pl.BlockSpec((16, _HC), lambda i: (0, 0)),
        ],
        out_specs=pl.BlockSpec((bn, _HC), lo),
        out_shape=jax.ShapeDtypeStruct((_N, _HC), _f32),
    )(xn, xn, x0f, x0f, MselA, MselB)


def kernel(X, vertex, edges, W, att_e):
    npad = _PADNNZ - _NNZ
    padz = jnp.zeros((npad,), jnp.int32)
    pade = jnp.full((npad,), _E, jnp.int32)
    padv = jnp.full((npad,), _N, jnp.int32)

    z64 = jnp.zeros((_E, _HALF), _f32)
    z16 = jnp.zeros((_E, 16), _f32)
    ones16 = jnp.ones((_CHUNK, 16), _f32)

    eidx2 = jnp.concatenate([edges, pade]).reshape(_NCORES * _NSUB, _NCH // 2,
                                                  _CHUNK)
    cnt = _phase_b2(eidx2, z16, ones16)

    x0f = _phase_a(X, W)

    vidx_b = jnp.concatenate([vertex, padz, vertex + _N, padz]) \
        .reshape(_NCORES * _NSUB, _NCH, _CHUNK)
    eidx_b = jnp.concatenate([edges, pade]).reshape(_NSUB, _NCH, _CHUNK)

    xe_sum = _phase_b(x0f, vidx_b, eidx_b, z64)

    attf = att_e.reshape(_H, _C)
    A = jnp.zeros((_NCORES, _HALF, _HALF), _f32)
    A = A.at[0, 0:32, 0].set(attf[0]).at[0, 32:64, 1].set(attf[1])
    A = A.at[1, 0:32, 0].set(attf[2]).at[1, 32:64, 1].set(attf[3])
    M2 = jnp.zeros((_HALF, _HALF), _f32).at[0, 0:32].set(1.0).at[1, 32:64].set(1.0)

    ge = _phase_c(xe_sum, cnt, A, M2)

    eidx_d = jnp.concatenate([edges, padz, edges + _E, padz]) \
        .reshape(_NCORES * _NSUB, _NCH, _CHUNK)
    vidx_d = jnp.concatenate([vertex, padv]).reshape(_NSUB, _NCH, _CHUNK)
    z80 = jnp.zeros((_N, _W80), _f32)

    xn = _phase_d(ge, vidx_d, eidx_d, z80)

    MselA = jnp.zeros((16, _HC), _f32).at[0, 0:32].set(1.0).at[1, 32:64].set(1.0)
    MselB = jnp.zeros((16, _HC), _f32).at[0, 64:96].set(1.0).at[1, 96:128].set(1.0)

    return _phase_e(xn, x0f, MselA, MselB)


# speedup vs baseline: 1.0505x; 1.0505x over previous
import jax
import jax.numpy as jnp
from jax import lax
from jax.experimental import pallas as pl
from jax.experimental.pallas import tpu as pltpu
from jax.experimental.pallas import tpu_sc as plsc

_N = 10000
_E = 20000
_NNZ = 320000
_H = 4
_C = 32
_HC = _H * _C
_HALF = _HC // 2
_NCORES = 2
_NSUB = 16
_CHUNK = 128
_NCH = 160
_NB_B = 2
_NB_D = 4
_NSTAGE = 4
_CPS = _NCH // _NSTAGE
_PPS = _NCH * _CHUNK
_PADNNZ = _NSUB * _PPS
_W80 = _HALF + 16
_f32 = jnp.float32

_HIGH = lax.Precision.HIGHEST


def _mm_body(x_ref, w_ref, o_ref):
    o_ref[...] = lax.dot_general(
        x_ref[...], w_ref[...], (((1,), (1,)), ((), ())),
        preferred_element_type=_f32, precision=_HIGH)


def _phase_a(X, W):
    bn = 2000
    nb = _N // bn
    return pl.pallas_call(
        _mm_body,
        grid=(_NCORES, nb),
        in_specs=[
            pl.BlockSpec((bn, _HC), lambda c, i: (i, 0)),
            pl.BlockSpec((_HALF, _HC), lambda c, i: (c, 0)),
        ],
        out_specs=pl.BlockSpec((bn, _HALF), lambda c, i, _nb=nb: (c * _nb + i, 0)),
        out_shape=jax.ShapeDtypeStruct((_NCORES * _N, _HALF), _f32),
    )(X, W)


def _sliced_copy(s, total, src, dst, src_off=0, dst_off=0):
    base = (total // _NSUB) // 8 * 8
    tail = total - _NSUB * base

    def cp(r0, nrows):
        so = pl.multiple_of(src_off + r0, 8)
        do = pl.multiple_of(dst_off + r0, 8)
        pltpu.sync_copy(src.at[pl.ds(so, nrows)], dst.at[pl.ds(do, nrows)])

    cp(s * base, base)
    if tail:
        @pl.when(s == _NSUB - 1)
        def _():
            cp(_NSUB * base, tail)


def _phase_b_body(x0f, vidx, eidx, z64, z16, ones16,
                  xe_out, cnt_out,
                  vall, eall, rb0, rb1, obuf,
                  g0, g1, s0, s1, c0, c1,
                  xe_sh, cnt_sh):
    c = lax.axis_index("c")
    s = lax.axis_index("s")
    w = c * _NSUB + s
    rbufs = [rb0, rb1]
    gsems = [g0, g1]
    ssems = [s0, s1]
    csems = [c0, c1]

    _sliced_copy(s, _E, z64, xe_sh)
    _sliced_copy(s, _E, z16, cnt_sh)
    pltpu.sync_copy(ones16, obuf)
    plsc.subcore_barrier()

    def fire_gather(j, b):
        pltpu.async_copy(x0f.at[vall.at[j]], rbufs[b], gsems[b])

    def wait_gather(j, b):
        pltpu.make_async_copy(x0f.at[vall.at[j]], rbufs[b], gsems[b]).wait()

    def fire_cnt(j, b):
        pltpu.async_copy(obuf, cnt_sh.at[eall.at[j]], csems[b], add=True)

    def wait_cnt(j, b):
        pltpu.make_async_copy(obuf, cnt_sh.at[eall.at[j]], csems[b]).wait()

    def step(i, b, last):
        j = i * _NB_B + b
        wait_gather(j, b)
        wait_cnt(j, b)
        sc = pltpu.async_copy(rbufs[b], xe_sh.at[eall.at[j]], ssems[b],
                              add=True)
        sc.wait()
        if not last:
            fire_gather(j + _NB_B, b)
            fire_cnt(j + _NB_B, b)

    gps = _CPS // _NB_B
    for st in range(_NSTAGE):
        pltpu.sync_copy(vidx.at[w].at[pl.ds(st * _CPS, _CPS)], vall)
        pltpu.sync_copy(eidx.at[s].at[pl.ds(st * _CPS, _CPS)], eall)
        for b in range(_NB_B):
            fire_gather(b, b)
            fire_cnt(b, b)

        def group(i, carry):
            for b in range(_NB_B):
                step(i, b, False)
            return carry

        lax.fori_loop(0, gps - 1, group, 0)
        for b in range(_NB_B):
            step(gps - 1, b, True)

    plsc.subcore_barrier()
    _sliced_copy(s, _E, xe_sh, xe_out, dst_off=c * _E)
    _sliced_copy(s, _E, cnt_sh, cnt_out, dst_off=c * _E)


def _phase_b(x0f, vidx, eidx, z64, z16, ones16):
    mesh = plsc.VectorSubcoreMesh(core_axis_name="c", subcore_axis_name="s")
    f = pl.kernel(
        _phase_b_body,
        out_type=(jax.ShapeDtypeStruct((_NCORES * _E, _HALF), _f32),
                  jax.ShapeDtypeStruct((_NCORES * _E, 16), _f32)),
        mesh=mesh,
        scratch_types=(
            [pltpu.VMEM((_CPS, _CHUNK), jnp.int32)] * 2
            + [pltpu.VMEM((_CHUNK, _HALF), _f32)] * _NB_B
            + [pltpu.VMEM((_CHUNK, 16), _f32)]
            + [pltpu.SemaphoreType.DMA] * (3 * _NB_B)
            + [pltpu.VMEM_SHARED((_E + 8, _HALF), _f32),
               pltpu.VMEM_SHARED((_E + 8, 16), _f32)]
        ),
        compiler_params=pltpu.CompilerParams(use_tc_tiling_on_sc=False),
    )
    return f(x0f, vidx, eidx, z64, z16, ones16)


def _phase_c_body(xe_ref, cnt_ref, a_ref, m2_ref, ge_ref, g16_ref):
    xs = xe_ref[...]
    cnt = cnt_ref[:, 0:1]
    xe = xs / jnp.maximum(cnt, 1.0)
    al = lax.dot_general(xe, a_ref[0], (((1,), (0,)), ((), ())),
                         preferred_element_type=_f32, precision=_HIGH)
    lr = jnp.where(al >= 0.0, al, al * 0.01)
    g = jnp.exp(lr)
    gb = lax.dot_general(g, m2_ref[...], (((1,), (0,)), ((), ())),
                         preferred_element_type=_f32, precision=_HIGH)
    ge_ref[...] = gb * xe
    g16_ref[...] = g[:, :16]


def _phase_c(xe_sum, cnt, A, M2):
    be = 2000
    nb = _E // be
    return pl.pallas_call(
        _phase_c_body,
        grid=(_NCORES, nb),
        in_specs=[
            pl.BlockSpec((be, _HALF), lambda c, i, _nb=nb: (c * _nb + i, 0)),
            pl.BlockSpec((be, 16), lambda c, i: (i, 0)),
            pl.BlockSpec((1, _HALF, _HALF), lambda c, i: (c, 0, 0)),
            pl.BlockSpec((_HALF, _HALF), lambda c, i: (0, 0)),
        ],
        out_specs=[
            pl.BlockSpec((be, _HALF), lambda c, i, _nb=nb: (c * _nb + i, 0)),
            pl.BlockSpec((be, 16), lambda c, i, _nb=nb: (c * _nb + i, 0)),
        ],
        out_shape=(jax.ShapeDtypeStruct((_NCORES * _E, _HALF), _f32),
                   jax.ShapeDtypeStruct((_NCORES * _E, 16), _f32)),
    )(xe_sum, cnt, A, M2)


def _phase_d_body(gef, g16f, vidx, eidx, z64, z16,
                  xn_out, den_out,
                  vall, eall, rb0, rb1, rb2, rb3, qb0, qb1, qb2, qb3,
                  g0, g1, g2, g3, s0, s1, s2, s3,
                  xn_sh, den_sh):
    c = lax.axis_index("c")
    s = lax.axis_index("s")
    w = c * _NSUB + s
    rbufs = [rb0, rb1, rb2, rb3]
    qbufs = [qb0, qb1, qb2, qb3]
    gsems = [g0, g1, g2, g3]
    ssems = [s0, s1, s2, s3]

    _sliced_copy(s, _N, z64, xn_sh)
    _sliced_copy(s, _N, z16, den_sh)
    plsc.subcore_barrier()

    def fire_gathers(j, b):
        pltpu.async_copy(gef.at[eall.at[j]], rbufs[b], gsems[b])
        pltpu.async_copy(g16f.at[eall.at[j]], qbufs[b], gsems[b])

    def wait_gathers(j, b):
        pltpu.make_async_copy(gef.at[eall.at[j]], rbufs[b], gsems[b]).wait()
        pltpu.make_async_copy(g16f.at[eall.at[j]], qbufs[b], gsems[b]).wait()

    def step(i, b, last):
        j = i * _NB_D + b
        wait_gathers(j, b)
        sc1 = pltpu.async_copy(rbufs[b], xn_sh.at[vall.at[j]], ssems[b],
                               add=True)
        sc2 = pltpu.async_copy(qbufs[b], den_sh.at[vall.at[j]], ssems[b],
                               add=True)
        sc1.wait()
        sc2.wait()
        if not last:
            fire_gathers(j + _NB_D, b)

    gps = _CPS // _NB_D
    for st in range(_NSTAGE):
        pltpu.sync_copy(eidx.at[w].at[pl.ds(st * _CPS, _CPS)], eall)
        pltpu.sync_copy(vidx.at[s].at[pl.ds(st * _CPS, _CPS)], vall)
        for b in range(_NB_D):
            fire_gathers(b, b)

        def group(i, carry):
            for b in range(_NB_D):
                step(i, b, False)
            return carry

        lax.fori_loop(0, gps - 1, group, 0)
        for b in range(_NB_D):
            step(gps - 1, b, True)

    plsc.subcore_barrier()
    _sliced_copy(s, _N, xn_sh, xn_out, dst_off=c * _N)
    _sliced_copy(s, _N, den_sh, den_out, dst_off=c * _N)


def _phase_d(gef, g16f, vidx, eidx, z64, z16):
    mesh = plsc.VectorSubcoreMesh(core_axis_name="c", subcore_axis_name="s")
    f = pl.kernel(
        _phase_d_body,
        out_type=(jax.ShapeDtypeStruct((_NCORES * _N, _HALF), _f32),
                  jax.ShapeDtypeStruct((_NCORES * _N, 16), _f32)),
        mesh=mesh,
        scratch_types=(
            [pltpu.VMEM((_CPS, _CHUNK), jnp.int32)] * 2
            + [pltpu.VMEM((_CHUNK, _HALF), _f32)] * _NB_D
            + [pltpu.VMEM((_CHUNK, 16), _f32)] * _NB_D
            + [pltpu.SemaphoreType.DMA] * (2 * _NB_D)
            + [pltpu.VMEM_SHARED((_N + 8, _HALF), _f32),
               pltpu.VMEM_SHARED((_N + 8, 16), _f32)]
        ),
        compiler_params=pltpu.CompilerParams(use_tc_tiling_on_sc=False),
    )
    return f(gef, g16f, vidx, eidx, z64, z16)


def _phase_e_body(xn0, xn1, dn0, dn1, x00, x01, msa, msb, o_ref):
    denb = (lax.dot_general(dn0[...], msa[...], (((1,), (0,)), ((), ())),
                            preferred_element_type=_f32, precision=_HIGH)
            + lax.dot_general(dn1[...], msb[...], (((1,), (0,)), ((), ())),
                              preferred_element_type=_f32, precision=_HIGH))
    num = jnp.concatenate([xn0[...], xn1[...]], axis=1)
    x0 = jnp.concatenate([x00[...], x01[...]], axis=1)
    o_ref[...] = num / (denb + 1e-16) + x0


def _phase_e(xn, den, x0f, MselA, MselB):
    bn = 2000
    nb = _N // bn

    def lo(i):
        return (i, 0)

    def hi(i, _nb=nb):
        return (_nb + i, 0)

    return pl.pallas_call(
        _phase_e_body,
        grid=(nb,),
        in_specs=[
            pl.BlockSpec((bn, _HALF), lo),
            pl.BlockSpec((bn, _HALF), hi),
            pl.BlockSpec((bn, 16), lo),
            pl.BlockSpec((bn, 16), hi),
            pl.BlockSpec((bn, _HALF), lo),
            pl.BlockSpec((bn, _HALF), hi),
            pl.BlockSpec((16, _HC), lambda i: (0, 0)),
            pl.BlockSpec((16, _HC), lambda i: (0, 0)),
        ],
        out_specs=pl.BlockSpec((bn, _HC), lo),
        out_shape=jax.ShapeDtypeStruct((_N, _HC), _f32),
    )(xn, xn, den, den, x0f, x0f, MselA, MselB)


def kernel(X, vertex, edges, W, att_e):
    npad = _PADNNZ - _NNZ
    padz = jnp.zeros((npad,), jnp.int32)
    pade = jnp.full((npad,), _E, jnp.int32)
    padv = jnp.full((npad,), _N, jnp.int32)

    z64 = jnp.zeros((_E, _HALF), _f32)
    z16 = jnp.zeros((_E, 16), _f32)
    ones16 = jnp.ones((_CHUNK, 16), _f32)

    x0f = _phase_a(X, W)

    vidx_b = jnp.concatenate([vertex, padz, vertex + _N, padz]) \
        .reshape(_NCORES * _NSUB, _NCH, _CHUNK)
    eidx_b = jnp.concatenate([edges, pade]).reshape(_NSUB, _NCH, _CHUNK)

    xe_sum, cnt = _phase_b(x0f, vidx_b, eidx_b, z64, z16, ones16)

    attf = att_e.reshape(_H, _C)
    A = jnp.zeros((_NCORES, _HALF, _HALF), _f32)
    A = A.at[0, 0:32, 0].set(attf[0]).at[0, 32:64, 1].set(attf[1])
    A = A.at[1, 0:32, 0].set(attf[2]).at[1, 32:64, 1].set(attf[3])
    M2 = jnp.zeros((_HALF, _HALF), _f32).at[0, 0:32].set(1.0).at[1, 32:64].set(1.0)

    ge, g16 = _phase_c(xe_sum, cnt, A, M2)

    eidx_d = jnp.concatenate([edges, padz, edges + _E, padz]) \
        .reshape(_NCORES * _NSUB, _NCH, _CHUNK)
    vidx_d = jnp.concatenate([vertex, padv]).reshape(_NSUB, _NCH, _CHUNK)

    xn, den = _phase_d(ge, g16, vidx_d, eidx_d, z64, z16)

    MselA = jnp.zeros((16, _HC), _f32).at[0, 0:32].set(1.0).at[1, 32:64].set(1.0)
    MselB = jnp.zeros((16, _HC), _f32).at[0, 64:96].set(1.0).at[1, 96:128].set(1.0)

    return _phase_e(xn, den, x0f, MselA, MselB)


# speedup vs baseline: 1.2544x; 1.1941x over previous
import jax
import jax.numpy as jnp
from jax import lax
from jax.experimental import pallas as pl
from jax.experimental.pallas import tpu as pltpu
from jax.experimental.pallas import tpu_sc as plsc

_N = 10000
_E = 20000
_NNZ = 320000
_H = 4
_C = 32
_HC = _H * _C
_HALF = _HC // 2
_NCORES = 2
_NSUB = 16
_CHUNK = 128
_NCH = 160
_NB_B = 2
_NB_D = 4
_NSTAGE = 4
_CPS = _NCH // _NSTAGE
_PPS = _NCH * _CHUNK
_PADNNZ = _NSUB * _PPS
_W80 = _HALF + 16
_f32 = jnp.float32

_HIGH = lax.Precision.HIGHEST


def _mm_body(x_ref, w_ref, o_ref):
    o_ref[...] = lax.dot_general(
        x_ref[...], w_ref[...], (((1,), (1,)), ((), ())),
        preferred_element_type=_f32, precision=_HIGH)


def _phase_a(X, W):
    bn = 2000
    nb = _N // bn
    return pl.pallas_call(
        _mm_body,
        grid=(_NCORES, nb),
        in_specs=[
            pl.BlockSpec((bn, _HC), lambda c, i: (i, 0)),
            pl.BlockSpec((_HALF, _HC), lambda c, i: (c, 0)),
        ],
        out_specs=pl.BlockSpec((bn, _HALF), lambda c, i, _nb=nb: (c * _nb + i, 0)),
        out_shape=jax.ShapeDtypeStruct((_NCORES * _N, _HALF), _f32),
    )(X, W)


def _sliced_copy(s, total, src, dst, src_off=0, dst_off=0):
    base = (total // _NSUB) // 8 * 8
    tail = total - _NSUB * base

    def cp(r0, nrows):
        so = pl.multiple_of(src_off + r0, 8)
        do = pl.multiple_of(dst_off + r0, 8)
        pltpu.sync_copy(src.at[pl.ds(so, nrows)], dst.at[pl.ds(do, nrows)])

    cp(s * base, base)
    if tail:
        @pl.when(s == _NSUB - 1)
        def _():
            cp(_NSUB * base, tail)


def _phase_b2_body(eidx, z16, ones16, cnt_out, eall, obuf, csem, cnt_sh):
    c = lax.axis_index("c")
    s = lax.axis_index("s")
    w = c * _NSUB + s
    _sliced_copy(s, _E, z16, cnt_sh)
    pltpu.sync_copy(ones16, obuf)
    plsc.subcore_barrier()

    def wait_cnt(j):
        pltpu.make_async_copy(obuf, cnt_sh.at[eall.at[j]], csem).wait()

    for st in range(2):
        pltpu.sync_copy(eidx.at[w].at[pl.ds(st * _CPS, _CPS)], eall)

        def chunk(j, carry):
            pltpu.async_copy(obuf, cnt_sh.at[eall.at[j]], csem, add=True)

            @pl.when(j >= 4)
            def _():
                wait_cnt(j - 4)
            return carry

        lax.fori_loop(0, _CPS, chunk, 0)
        for j in range(_CPS - 4, _CPS):
            wait_cnt(j)

    plsc.subcore_barrier()
    _sliced_copy(s, _E, cnt_sh, cnt_out, dst_off=c * _E)


def _phase_b2(eidx2, z16, ones16):
    mesh = plsc.VectorSubcoreMesh(core_axis_name="c", subcore_axis_name="s")
    f = pl.kernel(
        _phase_b2_body,
        out_type=jax.ShapeDtypeStruct((_NCORES * _E, 16), _f32),
        mesh=mesh,
        scratch_types=[
            pltpu.VMEM((_CPS, _CHUNK), jnp.int32),
            pltpu.VMEM((_CHUNK, 16), _f32),
            pltpu.SemaphoreType.DMA,
            pltpu.VMEM_SHARED((_E + 8, 16), _f32),
        ],
        compiler_params=pltpu.CompilerParams(use_tc_tiling_on_sc=False),
    )
    return f(eidx2, z16, ones16)


def _phase_b_body(x0f, vidx, eidx, z64,
                  xe_out,
                  vall, eall, rbuf, gsem, ssem,
                  tbl_sh, xe_sh):
    c = lax.axis_index("c")
    s = lax.axis_index("s")
    _sliced_copy(s, _E, z64, xe_sh)
    _sliced_copy(s, _N, x0f, tbl_sh, src_off=c * _N)
    plsc.subcore_barrier()

    cps = 8
    for st in range(_NCH // cps):
        pltpu.sync_copy(vidx.at[s].at[pl.ds(st * cps, cps)], vall)
        pltpu.sync_copy(eidx.at[s].at[pl.ds(st * cps, cps)], eall)

        def chunk(j, carry):
            pltpu.async_copy(tbl_sh.at[vall.at[j]], rbuf, gsem).wait()
            pltpu.async_copy(rbuf, xe_sh.at[eall.at[j]], ssem, add=True).wait()
            return carry

        lax.fori_loop(0, cps, chunk, 0)

    plsc.subcore_barrier()
    _sliced_copy(s, _E, xe_sh, xe_out, dst_off=c * _E)


def _phase_b(x0f, vidx, eidx, z64):
    mesh = plsc.VectorSubcoreMesh(core_axis_name="c", subcore_axis_name="s")
    f = pl.kernel(
        _phase_b_body,
        out_type=jax.ShapeDtypeStruct((_NCORES * _E, _HALF), _f32),
        mesh=mesh,
        scratch_types=(
            [pltpu.VMEM((8, _CHUNK), jnp.int32)] * 2
            + [pltpu.VMEM((_CHUNK, _HALF), _f32)]
            + [pltpu.SemaphoreType.DMA] * 2
            + [pltpu.VMEM_SHARED((_N, _HALF), _f32),
               pltpu.VMEM_SHARED((_E + 8, _HALF), _f32)]
        ),
        compiler_params=pltpu.CompilerParams(use_tc_tiling_on_sc=False),
    )
    return f(x0f, vidx, eidx, z64)


def _phase_c_body(xe_ref, cnt0_ref, cnt1_ref, a_ref, m2_ref, ge_ref, g16_ref):
    xs = xe_ref[...]
    cnt = cnt0_ref[:, 0:1] + cnt1_ref[:, 0:1]
    xe = xs / jnp.maximum(cnt, 1.0)
    al = lax.dot_general(xe, a_ref[0], (((1,), (0,)), ((), ())),
                         preferred_element_type=_f32, precision=_HIGH)
    lr = jnp.where(al >= 0.0, al, al * 0.01)
    g = jnp.exp(lr)
    gb = lax.dot_general(g, m2_ref[...], (((1,), (0,)), ((), ())),
                         preferred_element_type=_f32, precision=_HIGH)
    ge_ref[...] = gb * xe
    g16_ref[...] = g[:, :16]


def _phase_c(xe_sum, cnt, A, M2):
    be = 2000
    nb = _E // be
    return pl.pallas_call(
        _phase_c_body,
        grid=(_NCORES, nb),
        in_specs=[
            pl.BlockSpec((be, _HALF), lambda c, i, _nb=nb: (c * _nb + i, 0)),
            pl.BlockSpec((be, 16), lambda c, i: (i, 0)),
            pl.BlockSpec((be, 16), lambda c, i, _nb=nb: (_nb + i, 0)),
            pl.BlockSpec((1, _HALF, _HALF), lambda c, i: (c, 0, 0)),
            pl.BlockSpec((_HALF, _HALF), lambda c, i: (0, 0)),
        ],
        out_specs=[
            pl.BlockSpec((be, _HALF), lambda c, i, _nb=nb: (c * _nb + i, 0)),
            pl.BlockSpec((be, 16), lambda c, i, _nb=nb: (c * _nb + i, 0)),
        ],
        out_shape=(jax.ShapeDtypeStruct((_NCORES * _E, _HALF), _f32),
                   jax.ShapeDtypeStruct((_NCORES * _E, 16), _f32)),
    )(xe_sum, cnt, cnt, A, M2)


def _phase_d_body(gef, g16f, vidx, eidx, z64, z16,
                  xn_out, den_out,
                  vall, eall, rb0, rb1, rb2, rb3, qb0, qb1, qb2, qb3,
                  g0, g1, g2, g3, s0, s1, s2, s3,
                  xn_sh, den_sh):
    c = lax.axis_index("c")
    s = lax.axis_index("s")
    w = c * _NSUB + s
    rbufs = [rb0, rb1, rb2, rb3]
    qbufs = [qb0, qb1, qb2, qb3]
    gsems = [g0, g1, g2, g3]
    ssems = [s0, s1, s2, s3]

    _sliced_copy(s, _N, z64, xn_sh)
    _sliced_copy(s, _N, z16, den_sh)
    plsc.subcore_barrier()

    def fire_gathers(j, b):
        pltpu.async_copy(gef.at[eall.at[j]], rbufs[b], gsems[b])
        pltpu.async_copy(g16f.at[eall.at[j]], qbufs[b], gsems[b])

    def wait_gathers(j, b):
        pltpu.make_async_copy(gef.at[eall.at[j]], rbufs[b], gsems[b]).wait()
        pltpu.make_async_copy(g16f.at[eall.at[j]], qbufs[b], gsems[b]).wait()

    def step(i, b, last):
        j = i * _NB_D + b
        wait_gathers(j, b)
        sc1 = pltpu.async_copy(rbufs[b], xn_sh.at[vall.at[j]], ssems[b],
                               add=True)
        sc2 = pltpu.async_copy(qbufs[b], den_sh.at[vall.at[j]], ssems[b],
                               add=True)
        sc1.wait()
        sc2.wait()
        if not last:
            fire_gathers(j + _NB_D, b)

    gps = _CPS // _NB_D
    for st in range(_NSTAGE):
        pltpu.sync_copy(eidx.at[w].at[pl.ds(st * _CPS, _CPS)], eall)
        pltpu.sync_copy(vidx.at[s].at[pl.ds(st * _CPS, _CPS)], vall)
        for b in range(_NB_D):
            fire_gathers(b, b)

        def group(i, carry):
            for b in range(_NB_D):
                step(i, b, False)
            return carry

        lax.fori_loop(0, gps - 1, group, 0)
        for b in range(_NB_D):
            step(gps - 1, b, True)

    plsc.subcore_barrier()
    _sliced_copy(s, _N, xn_sh, xn_out, dst_off=c * _N)
    _sliced_copy(s, _N, den_sh, den_out, dst_off=c * _N)


def _phase_d(gef, g16f, vidx, eidx, z64, z16):
    mesh = plsc.VectorSubcoreMesh(core_axis_name="c", subcore_axis_name="s")
    f = pl.kernel(
        _phase_d_body,
        out_type=(jax.ShapeDtypeStruct((_NCORES * _N, _HALF), _f32),
                  jax.ShapeDtypeStruct((_NCORES * _N, 16), _f32)),
        mesh=mesh,
        scratch_types=(
            [pltpu.VMEM((_CPS, _CHUNK), jnp.int32)] * 2
            + [pltpu.VMEM((_CHUNK, _HALF), _f32)] * _NB_D
            + [pltpu.VMEM((_CHUNK, 16), _f32)] * _NB_D
            + [pltpu.SemaphoreType.DMA] * (2 * _NB_D)
            + [pltpu.VMEM_SHARED((_N + 8, _HALF), _f32),
               pltpu.VMEM_SHARED((_N + 8, 16), _f32)]
        ),
        compiler_params=pltpu.CompilerParams(use_tc_tiling_on_sc=False),
    )
    return f(gef, g16f, vidx, eidx, z64, z16)


def _phase_e_body(xn0, xn1, dn0, dn1, x00, x01, msa, msb, o_ref):
    denb = (lax.dot_general(dn0[...], msa[...], (((1,), (0,)), ((), ())),
                            preferred_element_type=_f32, precision=_HIGH)
            + lax.dot_general(dn1[...], msb[...], (((1,), (0,)), ((), ())),
                              preferred_element_type=_f32, precision=_HIGH))
    num = jnp.concatenate([xn0[...], xn1[...]], axis=1)
    x0 = jnp.concatenate([x00[...], x01[...]], axis=1)
    o_ref[...] = num / (denb + 1e-16) + x0


def _phase_e(xn, den, x0f, MselA, MselB):
    bn = 2000
    nb = _N // bn

    def lo(i):
        return (i, 0)

    def hi(i, _nb=nb):
        return (_nb + i, 0)

    return pl.pallas_call(
        _phase_e_body,
        grid=(nb,),
        in_specs=[
            pl.BlockSpec((bn, _HALF), lo),
            pl.BlockSpec((bn, _HALF), hi),
            pl.BlockSpec((bn, 16), lo),
            pl.BlockSpec((bn, 16), hi),
            pl.BlockSpec((bn, _HALF), lo),
            pl.BlockSpec((bn, _HALF), hi),
            pl.BlockSpec((16, _HC), lambda i: (0, 0)),
            pl.BlockSpec((16, _HC), lambda i: (0, 0)),
        ],
        out_specs=pl.BlockSpec((bn, _HC), lo),
        out_shape=jax.ShapeDtypeStruct((_N, _HC), _f32),
    )(xn, xn, den, den, x0f, x0f, MselA, MselB)


def kernel(X, vertex, edges, W, att_e):
    npad = _PADNNZ - _NNZ
    padz = jnp.zeros((npad,), jnp.int32)
    pade = jnp.full((npad,), _E, jnp.int32)
    padv = jnp.full((npad,), _N, jnp.int32)

    z64 = jnp.zeros((_E, _HALF), _f32)
    z16 = jnp.zeros((_E, 16), _f32)
    ones16 = jnp.ones((_CHUNK, 16), _f32)

    eidx2 = jnp.concatenate([edges, pade]).reshape(_NCORES * _NSUB, _NCH // 2,
                                                  _CHUNK)
    cnt = _phase_b2(eidx2, z16, ones16)

    x0f = _phase_a(X, W)

    vidx_b = jnp.concatenate([vertex, padz]).reshape(_NSUB, _NCH, _CHUNK)
    eidx_b = jnp.concatenate([edges, pade]).reshape(_NSUB, _NCH, _CHUNK)

    xe_sum = _phase_b(x0f, vidx_b, eidx_b, z64)

    attf = att_e.reshape(_H, _C)
    A = jnp.zeros((_NCORES, _HALF, _HALF), _f32)
    A = A.at[0, 0:32, 0].set(attf[0]).at[0, 32:64, 1].set(attf[1])
    A = A.at[1, 0:32, 0].set(attf[2]).at[1, 32:64, 1].set(attf[3])
    M2 = jnp.zeros((_HALF, _HALF), _f32).at[0, 0:32].set(1.0).at[1, 32:64].set(1.0)

    ge, g16 = _phase_c(xe_sum, cnt, A, M2)

    eidx_d = jnp.concatenate([edges, padz, edges + _E, padz]) \
        .reshape(_NCORES * _NSUB, _NCH, _CHUNK)
    vidx_d = jnp.concatenate([vertex, padv]).reshape(_NSUB, _NCH, _CHUNK)

    xn, den = _phase_d(ge, g16, vidx_d, eidx_d, z64, z16)

    MselA = jnp.zeros((16, _HC), _f32).at[0, 0:32].set(1.0).at[1, 32:64].set(1.0)
    MselB = jnp.zeros((16, _HC), _f32).at[0, 64:96].set(1.0).at[1, 96:128].set(1.0)

    return _phase_e(xn, den, x0f, MselA, MselB)


# speedup vs baseline: 1.3474x; 1.0741x over previous
import jax
import jax.numpy as jnp
from jax import lax
from jax.experimental import pallas as pl
from jax.experimental.pallas import tpu as pltpu
from jax.experimental.pallas import tpu_sc as plsc

_N = 10000
_E = 20000
_NNZ = 320000
_H = 4
_C = 32
_HC = _H * _C
_HALF = _HC // 2
_NCORES = 2
_NSUB = 16
_CHUNK = 128
_NCH = 160
_NB_B = 2
_NB_D = 4
_NSTAGE = 4
_CPS = _NCH // _NSTAGE
_PPS = _NCH * _CHUNK
_PADNNZ = _NSUB * _PPS
_W80 = _HALF + 16
_f32 = jnp.float32

_HIGH = lax.Precision.HIGHEST


def _mm_body(x_ref, w_ref, o_ref):
    o_ref[...] = lax.dot_general(
        x_ref[...], w_ref[...], (((1,), (1,)), ((), ())),
        preferred_element_type=_f32, precision=_HIGH)


def _phase_a(X, W):
    bn = 2000
    nb = _N // bn
    return pl.pallas_call(
        _mm_body,
        grid=(_NCORES, nb),
        in_specs=[
            pl.BlockSpec((bn, _HC), lambda c, i: (i, 0)),
            pl.BlockSpec((_HALF, _HC), lambda c, i: (c, 0)),
        ],
        out_specs=pl.BlockSpec((bn, _HALF), lambda c, i, _nb=nb: (c * _nb + i, 0)),
        out_shape=jax.ShapeDtypeStruct((_NCORES * _N, _HALF), _f32),
    )(X, W)


def _sliced_copy(s, total, src, dst, src_off=0, dst_off=0):
    base = (total // _NSUB) // 8 * 8
    tail = total - _NSUB * base

    def cp(r0, nrows):
        so = pl.multiple_of(src_off + r0, 8)
        do = pl.multiple_of(dst_off + r0, 8)
        pltpu.sync_copy(src.at[pl.ds(so, nrows)], dst.at[pl.ds(do, nrows)])

    cp(s * base, base)
    if tail:
        @pl.when(s == _NSUB - 1)
        def _():
            cp(_NSUB * base, tail)


def _phase_b2_body(eidx, z16, ones16, cnt_out, eall, obuf, csem, cnt_sh):
    c = lax.axis_index("c")
    s = lax.axis_index("s")
    w = c * _NSUB + s
    _sliced_copy(s, _E, z16, cnt_sh)
    pltpu.sync_copy(ones16, obuf)
    plsc.subcore_barrier()

    def wait_cnt(j):
        pltpu.make_async_copy(obuf, cnt_sh.at[eall.at[j]], csem).wait()

    for st in range(2):
        pltpu.sync_copy(eidx.at[w].at[pl.ds(st * _CPS, _CPS)], eall)

        def chunk(j, carry):
            pltpu.async_copy(obuf, cnt_sh.at[eall.at[j]], csem, add=True)

            @pl.when(j >= 4)
            def _():
                wait_cnt(j - 4)
            return carry

        lax.fori_loop(0, _CPS, chunk, 0)
        for j in range(_CPS - 4, _CPS):
            wait_cnt(j)

    plsc.subcore_barrier()
    _sliced_copy(s, _E, cnt_sh, cnt_out, dst_off=c * _E)


def _phase_b2(eidx2, z16, ones16):
    mesh = plsc.VectorSubcoreMesh(core_axis_name="c", subcore_axis_name="s")
    f = pl.kernel(
        _phase_b2_body,
        out_type=jax.ShapeDtypeStruct((_NCORES * _E, 16), _f32),
        mesh=mesh,
        scratch_types=[
            pltpu.VMEM((_CPS, _CHUNK), jnp.int32),
            pltpu.VMEM((_CHUNK, 16), _f32),
            pltpu.SemaphoreType.DMA,
            pltpu.VMEM_SHARED((_E + 8, 16), _f32),
        ],
        compiler_params=pltpu.CompilerParams(use_tc_tiling_on_sc=False),
    )
    return f(eidx2, z16, ones16)


_BCH = 64
_BNCH = _PPS // _BCH
_BCPS = 16


def _phase_b_body(x0f, vidx, eidx, z64,
                  xe_out,
                  vall, eall, rb0, rb1, g0, g1, s0, s1,
                  tbl_sh, xe_sh):
    c = lax.axis_index("c")
    s = lax.axis_index("s")
    rbufs = [rb0, rb1]
    gsems = [g0, g1]
    ssems = [s0, s1]
    _sliced_copy(s, _E, z64, xe_sh)
    _sliced_copy(s, _N, x0f, tbl_sh, src_off=c * _N)
    plsc.subcore_barrier()

    def fire_gather(j, b):
        pltpu.async_copy(tbl_sh.at[vall.at[j]], rbufs[b], gsems[b])

    def wait_gather(j, b):
        pltpu.make_async_copy(tbl_sh.at[vall.at[j]], rbufs[b], gsems[b]).wait()

    def step(i, b, last):
        j = i * 2 + b
        wait_gather(j, b)
        pltpu.async_copy(rbufs[b], xe_sh.at[eall.at[j]], ssems[b],
                         add=True).wait()
        if not last:
            fire_gather(j + 2, b)

    for st in range(_BNCH // _BCPS):
        pltpu.sync_copy(vidx.at[s].at[pl.ds(st * _BCPS, _BCPS)], vall)
        pltpu.sync_copy(eidx.at[s].at[pl.ds(st * _BCPS, _BCPS)], eall)
        for b in range(2):
            fire_gather(b, b)

        def group(i, carry):
            for b in range(2):
                step(i, b, False)
            return carry

        lax.fori_loop(0, _BCPS // 2 - 1, group, 0)
        for b in range(2):
            step(_BCPS // 2 - 1, b, True)

    plsc.subcore_barrier()
    _sliced_copy(s, _E, xe_sh, xe_out, dst_off=c * _E)


def _phase_b(x0f, vidx, eidx, z64):
    mesh = plsc.VectorSubcoreMesh(core_axis_name="c", subcore_axis_name="s")
    f = pl.kernel(
        _phase_b_body,
        out_type=jax.ShapeDtypeStruct((_NCORES * _E, _HALF), _f32),
        mesh=mesh,
        scratch_types=(
            [pltpu.VMEM((_BCPS, _BCH), jnp.int32)] * 2
            + [pltpu.VMEM((_BCH, _HALF), _f32)] * 2
            + [pltpu.SemaphoreType.DMA] * 4
            + [pltpu.VMEM_SHARED((_N, _HALF), _f32),
               pltpu.VMEM_SHARED((_E + 8, _HALF), _f32)]
        ),
        compiler_params=pltpu.CompilerParams(use_tc_tiling_on_sc=False),
    )
    return f(x0f, vidx, eidx, z64)


def _phase_c_body(xe_ref, cnt0_ref, cnt1_ref, a_ref, m2_ref, ge_ref, g16_ref):
    xs = xe_ref[...]
    cnt = cnt0_ref[:, 0:1] + cnt1_ref[:, 0:1]
    xe = xs / jnp.maximum(cnt, 1.0)
    al = lax.dot_general(xe, a_ref[0], (((1,), (0,)), ((), ())),
                         preferred_element_type=_f32, precision=_HIGH)
    lr = jnp.where(al >= 0.0, al, al * 0.01)
    g = jnp.exp(lr)
    gb = lax.dot_general(g, m2_ref[...], (((1,), (0,)), ((), ())),
                         preferred_element_type=_f32, precision=_HIGH)
    ge_ref[...] = gb * xe
    g16_ref[...] = g[:, :16]


def _phase_c(xe_sum, cnt, A, M2):
    be = 2000
    nb = _E // be
    return pl.pallas_call(
        _phase_c_body,
        grid=(_NCORES, nb),
        in_specs=[
            pl.BlockSpec((be, _HALF), lambda c, i, _nb=nb: (c * _nb + i, 0)),
            pl.BlockSpec((be, 16), lambda c, i: (i, 0)),
            pl.BlockSpec((be, 16), lambda c, i, _nb=nb: (_nb + i, 0)),
            pl.BlockSpec((1, _HALF, _HALF), lambda c, i: (c, 0, 0)),
            pl.BlockSpec((_HALF, _HALF), lambda c, i: (0, 0)),
        ],
        out_specs=[
            pl.BlockSpec((be, _HALF), lambda c, i, _nb=nb: (c * _nb + i, 0)),
            pl.BlockSpec((be, 16), lambda c, i, _nb=nb: (c * _nb + i, 0)),
        ],
        out_shape=(jax.ShapeDtypeStruct((_NCORES * _E, _HALF), _f32),
                   jax.ShapeDtypeStruct((_NCORES * _E, 16), _f32)),
    )(xe_sum, cnt, cnt, A, M2)


def _phase_d_body(gef, g16f, vidx, eidx, z64, z16,
                  xn_out, den_out,
                  vall, eall, rb0, rb1, rb2, rb3, qb0, qb1, qb2, qb3,
                  g0, g1, g2, g3, s0, s1, s2, s3,
                  xn_sh, den_sh):
    c = lax.axis_index("c")
    s = lax.axis_index("s")
    w = c * _NSUB + s
    rbufs = [rb0, rb1, rb2, rb3]
    qbufs = [qb0, qb1, qb2, qb3]
    gsems = [g0, g1, g2, g3]
    ssems = [s0, s1, s2, s3]

    _sliced_copy(s, _N, z64, xn_sh)
    _sliced_copy(s, _N, z16, den_sh)
    plsc.subcore_barrier()

    def fire_gathers(j, b):
        pltpu.async_copy(gef.at[eall.at[j]], rbufs[b], gsems[b])
        pltpu.async_copy(g16f.at[eall.at[j]], qbufs[b], gsems[b])

    def wait_gathers(j, b):
        pltpu.make_async_copy(gef.at[eall.at[j]], rbufs[b], gsems[b]).wait()
        pltpu.make_async_copy(g16f.at[eall.at[j]], qbufs[b], gsems[b]).wait()

    def step(i, b, last):
        j = i * _NB_D + b
        wait_gathers(j, b)
        sc1 = pltpu.async_copy(rbufs[b], xn_sh.at[vall.at[j]], ssems[b],
                               add=True)
        sc2 = pltpu.async_copy(qbufs[b], den_sh.at[vall.at[j]], ssems[b],
                               add=True)
        sc1.wait()
        sc2.wait()
        if not last:
            fire_gathers(j + _NB_D, b)

    gps = _CPS // _NB_D
    for st in range(_NSTAGE):
        pltpu.sync_copy(eidx.at[w].at[pl.ds(st * _CPS, _CPS)], eall)
        pltpu.sync_copy(vidx.at[s].at[pl.ds(st * _CPS, _CPS)], vall)
        for b in range(_NB_D):
            fire_gathers(b, b)

        def group(i, carry):
            for b in range(_NB_D):
                step(i, b, False)
            return carry

        lax.fori_loop(0, gps - 1, group, 0)
        for b in range(_NB_D):
            step(gps - 1, b, True)

    plsc.subcore_barrier()
    _sliced_copy(s, _N, xn_sh, xn_out, dst_off=c * _N)
    _sliced_copy(s, _N, den_sh, den_out, dst_off=c * _N)


def _phase_d(gef, g16f, vidx, eidx, z64, z16):
    mesh = plsc.VectorSubcoreMesh(core_axis_name="c", subcore_axis_name="s")
    f = pl.kernel(
        _phase_d_body,
        out_type=(jax.ShapeDtypeStruct((_NCORES * _N, _HALF), _f32),
                  jax.ShapeDtypeStruct((_NCORES * _N, 16), _f32)),
        mesh=mesh,
        scratch_types=(
            [pltpu.VMEM((_CPS, _CHUNK), jnp.int32)] * 2
            + [pltpu.VMEM((_CHUNK, _HALF), _f32)] * _NB_D
            + [pltpu.VMEM((_CHUNK, 16), _f32)] * _NB_D
            + [pltpu.SemaphoreType.DMA] * (2 * _NB_D)
            + [pltpu.VMEM_SHARED((_N + 8, _HALF), _f32),
               pltpu.VMEM_SHARED((_N + 8, 16), _f32)]
        ),
        compiler_params=pltpu.CompilerParams(use_tc_tiling_on_sc=False),
    )
    return f(gef, g16f, vidx, eidx, z64, z16)


def _phase_e_body(xn0, xn1, dn0, dn1, x00, x01, msa, msb, o_ref):
    denb = (lax.dot_general(dn0[...], msa[...], (((1,), (0,)), ((), ())),
                            preferred_element_type=_f32, precision=_HIGH)
            + lax.dot_general(dn1[...], msb[...], (((1,), (0,)), ((), ())),
                              preferred_element_type=_f32, precision=_HIGH))
    num = jnp.concatenate([xn0[...], xn1[...]], axis=1)
    x0 = jnp.concatenate([x00[...], x01[...]], axis=1)
    o_ref[...] = num / (denb + 1e-16) + x0


def _phase_e(xn, den, x0f, MselA, MselB):
    bn = 2000
    nb = _N // bn

    def lo(i):
        return (i, 0)

    def hi(i, _nb=nb):
        return (_nb + i, 0)

    return pl.pallas_call(
        _phase_e_body,
        grid=(nb,),
        in_specs=[
            pl.BlockSpec((bn, _HALF), lo),
            pl.BlockSpec((bn, _HALF), hi),
            pl.BlockSpec((bn, 16), lo),
            pl.BlockSpec((bn, 16), hi),
            pl.BlockSpec((bn, _HALF), lo),
            pl.BlockSpec((bn, _HALF), hi),
            pl.BlockSpec((16, _HC), lambda i: (0, 0)),
            pl.BlockSpec((16, _HC), lambda i: (0, 0)),
        ],
        out_specs=pl.BlockSpec((bn, _HC), lo),
        out_shape=jax.ShapeDtypeStruct((_N, _HC), _f32),
    )(xn, xn, den, den, x0f, x0f, MselA, MselB)


def kernel(X, vertex, edges, W, att_e):
    npad = _PADNNZ - _NNZ
    padz = jnp.zeros((npad,), jnp.int32)
    pade = jnp.full((npad,), _E, jnp.int32)
    padv = jnp.full((npad,), _N, jnp.int32)

    z64 = jnp.zeros((_E, _HALF), _f32)
    z16 = jnp.zeros((_E, 16), _f32)
    ones16 = jnp.ones((_CHUNK, 16), _f32)

    eidx2 = jnp.concatenate([edges, pade]).reshape(_NCORES * _NSUB, _NCH // 2,
                                                  _CHUNK)
    cnt = _phase_b2(eidx2, z16, ones16)

    x0f = _phase_a(X, W)

    vidx_b = jnp.concatenate([vertex, padz]).reshape(_NSUB, _BNCH, _BCH)
    eidx_b = jnp.concatenate([edges, pade]).reshape(_NSUB, _BNCH, _BCH)

    xe_sum = _phase_b(x0f, vidx_b, eidx_b, z64)

    attf = att_e.reshape(_H, _C)
    A = jnp.zeros((_NCORES, _HALF, _HALF), _f32)
    A = A.at[0, 0:32, 0].set(attf[0]).at[0, 32:64, 1].set(attf[1])
    A = A.at[1, 0:32, 0].set(attf[2]).at[1, 32:64, 1].set(attf[3])
    M2 = jnp.zeros((_HALF, _HALF), _f32).at[0, 0:32].set(1.0).at[1, 32:64].set(1.0)

    ge, g16 = _phase_c(xe_sum, cnt, A, M2)

    eidx_d = jnp.concatenate([edges, padz, edges + _E, padz]) \
        .reshape(_NCORES * _NSUB, _NCH, _CHUNK)
    vidx_d = jnp.concatenate([vertex, padv]).reshape(_NSUB, _NCH, _CHUNK)

    xn, den = _phase_d(ge, g16, vidx_d, eidx_d, z64, z16)

    MselA = jnp.zeros((16, _HC), _f32).at[0, 0:32].set(1.0).at[1, 32:64].set(1.0)
    MselB = jnp.zeros((16, _HC), _f32).at[0, 64:96].set(1.0).at[1, 96:128].set(1.0)

    return _phase_e(xn, den, x0f, MselA, MselB)
